# baseline (speedup 1.0000x reference)
import functools

import jax
import jax.numpy as jnp
from jax import lax
from jax.experimental import pallas as pl
from jax.experimental.pallas import tpu as pltpu
from jax.experimental.pallas import tpu_sc as plsc

B = 16384
H = 64
HP = 2 * H
NC, NS = 2, 16
NW = NC * NS
BPW = B // NW
CHUNK = 128
NCHUNK = BPW // CHUNK


def _sc_gather_body(tab, idx_hbm, out, idx_v, rows_v, sem):
    wid = lax.axis_index("s") * NC + lax.axis_index("c")
    base = wid * BPW
    pltpu.sync_copy(idx_hbm.at[wid], idx_v)
    copies = []
    for j in range(NCHUNK):
        copies.append(pltpu.async_copy(
            tab.at[idx_v.at[pl.ds(j * CHUNK, CHUNK)]],
            rows_v.at[pl.ds(j * CHUNK, CHUNK)], sem))
    for c in copies:
        c.wait()
    pltpu.sync_copy(rows_v, out.at[pl.ds(base, BPW)])


@functools.lru_cache(maxsize=None)
def _make_sc_gather():
    mesh = plsc.VectorSubcoreMesh(
        core_axis_name="c", subcore_axis_name="s",
        num_cores=NC, num_subcores=NS)
    return pl.kernel(
        _sc_gather_body,
        out_type=jax.ShapeDtypeStruct((B, HP), jnp.bfloat16),
        mesh=mesh,
        scratch_types=[
            pltpu.VMEM((BPW,), jnp.int32),
            pltpu.VMEM((BPW, HP), jnp.bfloat16),
            pltpu.SemaphoreType.DMA,
        ],
        compiler_params=pltpu.CompilerParams(use_tc_tiling_on_sc=False),
    )


BC = 16384


def _repack_body(t, o):
    xt = jnp.transpose(t[...], (1, 0)).astype(jnp.bfloat16)
    o[...] = jnp.concatenate(
        [xt, jnp.zeros((BC, H), jnp.bfloat16)], axis=1)


_repack = pl.pallas_call(
    _repack_body,
    grid=(pl.cdiv(100000, BC),),
    in_specs=[pl.BlockSpec((H, BC), lambda i: (0, i))],
    out_specs=pl.BlockSpec((BC, HP), lambda i: (i, 0)),
    out_shape=jax.ShapeDtypeStruct((100000, HP), jnp.bfloat16),
)


BS = 2048


def _mlp_body(p, n, w1p, w1n, b1, w2, b2, w3r, b3, o):
    h = jnp.dot(p[:, :H].astype(jnp.float32), w1p[...],
                preferred_element_type=jnp.float32)
    h = h + jnp.dot(n[:, :H].astype(jnp.float32), w1n[...],
                    preferred_element_type=jnp.float32)
    h = jnp.maximum(h + b1[...], 0.0)
    h = jnp.maximum(
        jnp.dot(h, w2[...], preferred_element_type=jnp.float32) + b2[...], 0.0)
    o[...] = jnp.sum(h * w3r[...], axis=1, keepdims=True) + b3[...]


_mlp = pl.pallas_call(
    _mlp_body,
    grid=(B // BS,),
    in_specs=[
        pl.BlockSpec((BS, HP), lambda i: (i, 0)),
        pl.BlockSpec((BS, HP), lambda i: (i, 0)),
        pl.BlockSpec((H, H), lambda i: (0, 0)),
        pl.BlockSpec((H, H), lambda i: (0, 0)),
        pl.BlockSpec((1, H), lambda i: (0, 0)),
        pl.BlockSpec((H, H), lambda i: (0, 0)),
        pl.BlockSpec((1, H), lambda i: (0, 0)),
        pl.BlockSpec((1, H), lambda i: (0, 0)),
        pl.BlockSpec((1, 1), lambda i: (0, 0)),
    ],
    out_specs=pl.BlockSpec((BS, 1), lambda i: (i, 0)),
    out_shape=jax.ShapeDtypeStruct((B, 1), jnp.float32),
)


def kernel(x, emb_proton, emb_neutron, W1, b1, W2, b2, W3, b3):
    xi = x.astype(jnp.int32)
    idx_p = xi[:, 0].reshape(NW, BPW)
    idx_n = xi[:, 1].reshape(NW, BPW)
    ptab = _repack(emb_proton.T)
    ntab = _repack(emb_neutron.T)
    gather = _make_sc_gather()
    prows = gather(ptab, idx_p)
    nrows = gather(ntab, idx_n)
    return _mlp(prows, nrows, W1[:H], W1[H:], b1.reshape(1, H),
                W2, b2.reshape(1, H), W3.reshape(1, H), b3.reshape(1, 1))


# speedup vs baseline: 2.6862x; 2.6862x over previous
import functools

import jax
import jax.numpy as jnp
from jax import lax
from jax.experimental import pallas as pl
from jax.experimental.pallas import tpu as pltpu
from jax.experimental.pallas import tpu_sc as plsc

B = 16384
H = 64
HP = 2 * H
NC, NS = 2, 16
NW = NC * NS
BPW = B // NW
CHUNK = 128
NCHUNK = BPW // CHUNK


def _sc_gather_body(tab, idx_hbm, out, idx_v, rows_v, sem):
    wid = lax.axis_index("s") * NC + lax.axis_index("c")
    base = wid * BPW
    pltpu.sync_copy(idx_hbm.at[wid], idx_v)
    copies = []
    for j in range(NCHUNK):
        copies.append(pltpu.async_copy(
            tab.at[idx_v.at[pl.ds(j * CHUNK, CHUNK)]],
            rows_v.at[pl.ds(j * CHUNK, CHUNK)], sem))
    for c in copies:
        c.wait()
    pltpu.sync_copy(rows_v, out.at[pl.ds(base, BPW)])


@functools.lru_cache(maxsize=None)
def _make_sc_gather():
    mesh = plsc.VectorSubcoreMesh(
        core_axis_name="c", subcore_axis_name="s",
        num_cores=NC, num_subcores=NS)
    return pl.kernel(
        _sc_gather_body,
        out_type=jax.ShapeDtypeStruct((B, HP), jnp.float32),
        mesh=mesh,
        scratch_types=[
            pltpu.VMEM((BPW,), jnp.int32),
            pltpu.VMEM((BPW, HP), jnp.float32),
            pltpu.SemaphoreType.DMA,
        ],
        compiler_params=pltpu.CompilerParams(use_tc_tiling_on_sc=True),
    )


BC = 16384


def _repack_body(t, o):
    xt = jnp.transpose(t[...], (1, 0))
    o[...] = jnp.concatenate(
        [xt, jnp.zeros((BC, H), jnp.float32)], axis=1)


_repack = pl.pallas_call(
    _repack_body,
    grid=(pl.cdiv(100000, BC),),
    in_specs=[pl.BlockSpec((H, BC), lambda i: (0, i))],
    out_specs=pl.BlockSpec((BC, HP), lambda i: (i, 0)),
    out_shape=jax.ShapeDtypeStruct((100000, HP), jnp.float32),
)


BS = 2048


def _mlp_body(p, n, w1p, w1n, b1, w2, b2, w3r, b3, o):
    h = jnp.dot(p[:, :H], w1p[...], preferred_element_type=jnp.float32)
    h = h + jnp.dot(n[:, :H], w1n[...], preferred_element_type=jnp.float32)
    h = jnp.maximum(h + b1[...], 0.0)
    h = jnp.maximum(
        jnp.dot(h, w2[...], preferred_element_type=jnp.float32) + b2[...], 0.0)
    o[...] = jnp.sum(h * w3r[...], axis=1) + b3[0, 0]


_mlp = pl.pallas_call(
    _mlp_body,
    grid=(B // BS,),
    in_specs=[
        pl.BlockSpec((BS, HP), lambda i: (i, 0)),
        pl.BlockSpec((BS, HP), lambda i: (i, 0)),
        pl.BlockSpec((H, H), lambda i: (0, 0)),
        pl.BlockSpec((H, H), lambda i: (0, 0)),
        pl.BlockSpec((1, H), lambda i: (0, 0)),
        pl.BlockSpec((H, H), lambda i: (0, 0)),
        pl.BlockSpec((1, H), lambda i: (0, 0)),
        pl.BlockSpec((1, H), lambda i: (0, 0)),
        pl.BlockSpec((1, 1), lambda i: (0, 0)),
    ],
    out_specs=pl.BlockSpec((BS,), lambda i: (i,)),
    out_shape=jax.ShapeDtypeStruct((B,), jnp.float32),
)


def kernel(x, emb_proton, emb_neutron, W1, b1, W2, b2, W3, b3):
    xi = x.astype(jnp.int32)
    idx_p = xi[:, 0].reshape(NW, BPW)
    idx_n = xi[:, 1].reshape(NW, BPW)
    ptab = _repack(emb_proton.T)
    ntab = _repack(emb_neutron.T)
    gather = _make_sc_gather()
    prows = gather(ptab, idx_p)
    nrows = gather(ntab, idx_n)
    out = _mlp(prows, nrows, W1[:H], W1[H:], b1.reshape(1, H),
               W2, b2.reshape(1, H), W3.reshape(1, H), b3.reshape(1, 1))
    return out.reshape(B, 1)


# speedup vs baseline: 2.9832x; 1.1106x over previous
import functools

import jax
import jax.numpy as jnp
from jax import lax
from jax.experimental import pallas as pl
from jax.experimental.pallas import tpu as pltpu
from jax.experimental.pallas import tpu_sc as plsc

B = 16384
H = 64
HP = 2 * H
NC, NS = 2, 16
NW = NC * NS
BPW = B // NW
CHUNK = 128
NCHUNK = BPW // CHUNK


def _sc_gather_body(tab, idx_hbm, out, idx_v, rows_v, sem):
    wid = lax.axis_index("s") * NC + lax.axis_index("c")
    base = wid * BPW
    pltpu.sync_copy(idx_hbm.at[wid], idx_v)
    copies = []
    for j in range(NCHUNK):
        copies.append(pltpu.async_copy(
            tab.at[idx_v.at[pl.ds(j * CHUNK, CHUNK)]],
            rows_v.at[pl.ds(j * CHUNK, CHUNK)], sem))
    for c in copies:
        c.wait()
    pltpu.sync_copy(rows_v, out.at[pl.ds(base, BPW)])


@functools.lru_cache(maxsize=None)
def _make_sc_gather():
    mesh = plsc.VectorSubcoreMesh(
        core_axis_name="c", subcore_axis_name="s",
        num_cores=NC, num_subcores=NS)
    return pl.kernel(
        _sc_gather_body,
        out_type=jax.ShapeDtypeStruct((B, HP), jnp.float32),
        mesh=mesh,
        scratch_types=[
            pltpu.VMEM((BPW,), jnp.int32),
            pltpu.VMEM((BPW, HP), jnp.float32),
            pltpu.SemaphoreType.DMA,
        ],
        compiler_params=pltpu.CompilerParams(use_tc_tiling_on_sc=True),
    )


BC = 16384


def _repack_body(t, o):
    xt = jnp.transpose(t[...], (1, 0))
    o[...] = jnp.concatenate(
        [xt, jnp.zeros((BC, H), jnp.float32)], axis=1)


_repack = pl.pallas_call(
    _repack_body,
    grid=(pl.cdiv(100000, BC),),
    in_specs=[pl.BlockSpec((H, BC), lambda i: (0, i))],
    out_specs=pl.BlockSpec((BC, HP), lambda i: (i, 0)),
    out_shape=jax.ShapeDtypeStruct((100000, HP), jnp.float32),
)


BS = 4096


def _mlp_body(p, n, w1p, w1n, b1, w2, b2, w3r, b3, o):
    h = jnp.dot(p[:, :H], w1p[...], preferred_element_type=jnp.float32)
    h = h + jnp.dot(n[:, :H], w1n[...], preferred_element_type=jnp.float32)
    h = jnp.maximum(h + b1[...], 0.0)
    h = jnp.maximum(
        jnp.dot(h, w2[...], preferred_element_type=jnp.float32) + b2[...], 0.0)
    o[...] = jnp.sum(h * w3r[...], axis=1, keepdims=True) + b3[...]


_mlp = pl.pallas_call(
    _mlp_body,
    grid=(B // BS,),
    in_specs=[
        pl.BlockSpec((BS, HP), lambda i: (i, 0)),
        pl.BlockSpec((BS, HP), lambda i: (i, 0)),
        pl.BlockSpec((H, H), lambda i: (0, 0)),
        pl.BlockSpec((H, H), lambda i: (0, 0)),
        pl.BlockSpec((1, H), lambda i: (0, 0)),
        pl.BlockSpec((H, H), lambda i: (0, 0)),
        pl.BlockSpec((1, H), lambda i: (0, 0)),
        pl.BlockSpec((1, H), lambda i: (0, 0)),
        pl.BlockSpec((1, 1), lambda i: (0, 0)),
    ],
    out_specs=pl.BlockSpec((BS, 1), lambda i: (i, 0)),
    out_shape=jax.ShapeDtypeStruct((B, 1), jnp.float32),
)


def kernel(x, emb_proton, emb_neutron, W1, b1, W2, b2, W3, b3):
    xi = x.astype(jnp.int32)
    idx_p = xi[:, 0].reshape(NW, BPW)
    idx_n = xi[:, 1].reshape(NW, BPW)
    ptab = _repack(emb_proton.T)
    ntab = _repack(emb_neutron.T)
    gather = _make_sc_gather()
    prows = gather(ptab, idx_p)
    nrows = gather(ntab, idx_n)
    return _mlp(prows, nrows, W1[:H], W1[H:], b1.reshape(1, H),
                W2, b2.reshape(1, H), W3.reshape(1, H), b3.reshape(1, 1))
